# Initial kernel scaffold; baseline (speedup 1.0000x reference)
import jax
import jax.numpy as jnp
from jax.experimental import pallas as pl


def kernel(x, edge_index, W1, b1, W2, b2):
    raise NotImplementedError("write your pallas kernel here")


# speedup vs baseline: 8.3403x; 8.3403x over previous
import functools

import jax
import jax.numpy as jnp
from jax import lax
from jax.experimental import pallas as pl
from jax.experimental.pallas import tpu as pltpu
from jax.experimental.pallas import tpu_sc as plsc

N = 10000
E = 320000
D = 128
NC = 2
NS = 16
NW = NC * NS
WIN_K = 128
E_PAD = 323584

DWIN_N = E_PAD // NW // WIN_K
NREG = N + 8

AWIN_N = E_PAD // NS // WIN_K
NH = N // NC
PARKN = 256
ACC_ROWS = 5280
ACH = 40

MB = 2000
GRID = N // MB

_MESH = dict(core_axis_name="c", subcore_axis_name="s",
             num_cores=NC, num_subcores=NS)


def _sc_degree(dst3, zeros_reg, ones_w):
    @functools.partial(
        pl.kernel,
        out_type=jax.ShapeDtypeStruct((NW * NREG,), jnp.float32),
        mesh=plsc.VectorSubcoreMesh(**_MESH),
        scratch_types=[
            pltpu.VMEM((DWIN_N, WIN_K), jnp.int32),
            pltpu.VMEM((WIN_K,), jnp.float32),
            pltpu.VMEM((NREG,), jnp.float32),
            pltpu.VMEM_SHARED((NS * NREG,), jnp.float32),
        ],
    )
    def deg_kernel(dst_hbm, z_hbm, ones_hbm, out_hbm, dstv, onesv, vbuf, cnt):
        c = lax.axis_index("c")
        s = lax.axis_index("s")
        w = c * NS + s
        pltpu.sync_copy(dst_hbm.at[w], dstv)
        pltpu.sync_copy(ones_hbm, onesv)

        base = s * NREG

        def remap(j, carry):
            for k in range(WIN_K // 16):
                dstv[j, pl.ds(k * 16, 16)] = (
                    dstv[j, pl.ds(k * 16, 16)] + base)
            return carry

        lax.fori_loop(0, DWIN_N, remap, 0)

        pltpu.sync_copy(z_hbm, vbuf)
        pltpu.sync_copy(vbuf, cnt.at[pl.ds(s * NREG, NREG)])

        def body(j, carry):
            pltpu.sync_copy(onesv, cnt.at[dstv.at[j]], add=True)
            return carry

        lax.fori_loop(0, DWIN_N, body, 0)
        pltpu.sync_copy(cnt.at[pl.ds(s * NREG, NREG)], vbuf)
        pltpu.sync_copy(vbuf, out_hbm.at[pl.ds(w * NREG, NREG)])

    return deg_kernel(dst3, zeros_reg, ones_w)


def _sc_aggregate(hs, src3, dst3, zeros_rows):
    @functools.partial(
        pl.kernel,
        out_type=jax.ShapeDtypeStruct((N, D), jnp.float32),
        mesh=plsc.VectorSubcoreMesh(**_MESH),
        scratch_types=[
            pltpu.VMEM((AWIN_N, WIN_K), jnp.int32),
            pltpu.VMEM((AWIN_N, WIN_K), jnp.int32),
            pltpu.VMEM((WIN_K, D), jnp.float32),
            pltpu.VMEM((ACH, D), jnp.float32),
            pltpu.VMEM_SHARED((ACC_ROWS, D), jnp.float32),
            pltpu.SemaphoreType.DMA,
        ],
    )
    def agg_kernel(hs_hbm, src_hbm, dst_hbm, z_hbm, out_hbm,
                   srcv, dstv, buf, chbuf, acc, sem):
        c = lax.axis_index("c")
        s = lax.axis_index("s")
        pltpu.sync_copy(src_hbm.at[s], srcv)
        pltpu.sync_copy(dst_hbm.at[s], dstv)

        lane = lax.iota(jnp.int32, 16)
        base = c * NH

        def remap(jj, carry):
            for m in range(2):
                j = jj * 2 + m
                for k in range(WIN_K // 16):
                    d = dstv[j, pl.ds(k * 16, 16)]
                    local = d - base
                    ok = (local >= 0) & (local < NH)
                    park = NH + m * WIN_K + k * 16 + lane
                    dstv[j, pl.ds(k * 16, 16)] = jnp.where(ok, local, park)
            return carry

        lax.fori_loop(0, AWIN_N // 2, remap, 0)

        pltpu.sync_copy(z_hbm, chbuf)
        for k in range(-(-(ACC_ROWS // ACH) // NS)):
            ch = s + k * NS

            @pl.when(ch < ACC_ROWS // ACH)
            def _():
                pltpu.sync_copy(chbuf, acc.at[pl.ds(ch * ACH, ACH)])

        plsc.subcore_barrier()

        def body(j, carry):
            pltpu.async_copy(hs_hbm.at[srcv.at[j]], buf, sem).wait()
            pltpu.sync_copy(buf, acc.at[dstv.at[j]], add=True)
            return carry

        lax.fori_loop(0, AWIN_N, body, 0)
        plsc.subcore_barrier()

        for k in range(-(-(NH // ACH) // NS)):
            ch = s + k * NS

            @pl.when(ch < NH // ACH)
            def _():
                pltpu.sync_copy(acc.at[pl.ds(ch * ACH, ACH)], chbuf)
                pltpu.sync_copy(
                    chbuf, out_hbm.at[pl.ds(c * NH + ch * ACH, ACH)])

    return agg_kernel(hs, src3, dst3, zeros_rows)


def _tc_layer1(x, W1, degpt):
    def body(x_ref, w_ref, dp_ref, hs_ref):
        g = lax.rsqrt(1.0 + jnp.sum(dp_ref[...], axis=1, keepdims=True))
        hs_ref[...] = (
            jnp.dot(x_ref[...], w_ref[...], preferred_element_type=jnp.float32)
            * g
        )

    return pl.pallas_call(
        body,
        grid=(GRID,),
        in_specs=[
            pl.BlockSpec((MB, D), lambda i: (i, 0)),
            pl.BlockSpec((D, D), lambda i: (0, 0)),
            pl.BlockSpec((MB, NW), lambda i: (i, 0)),
        ],
        out_specs=pl.BlockSpec((MB, D), lambda i: (i, 0)),
        out_shape=jax.ShapeDtypeStruct((N, D), jnp.float32),
    )(x, W1, degpt)


def _tc_layer2(agg1, hs1, degpt, W2, b1):
    def body(a_ref, h_ref, dp_ref, w_ref, b_ref, out_ref):
        g = lax.rsqrt(1.0 + jnp.sum(dp_ref[...], axis=1, keepdims=True))
        t = (a_ref[...] + h_ref[...]) * g + b_ref[...]
        t = jnp.maximum(t, 0.0)
        out_ref[...] = (
            jnp.dot(t, w_ref[...], preferred_element_type=jnp.float32) * g
        )

    return pl.pallas_call(
        body,
        grid=(GRID,),
        in_specs=[
            pl.BlockSpec((MB, D), lambda i: (i, 0)),
            pl.BlockSpec((MB, D), lambda i: (i, 0)),
            pl.BlockSpec((MB, NW), lambda i: (i, 0)),
            pl.BlockSpec((D, D), lambda i: (0, 0)),
            pl.BlockSpec((1, D), lambda i: (0, 0)),
        ],
        out_specs=pl.BlockSpec((MB, D), lambda i: (i, 0)),
        out_shape=jax.ShapeDtypeStruct((N, D), jnp.float32),
    )(agg1, hs1, degpt, W2, b1)


def _tc_layer3(agg2, hs2, degpt, b2):
    def body(a_ref, h_ref, dp_ref, b_ref, out_ref):
        g = lax.rsqrt(1.0 + jnp.sum(dp_ref[...], axis=1, keepdims=True))
        out_ref[...] = (a_ref[...] + h_ref[...]) * g + b_ref[...]

    return pl.pallas_call(
        body,
        grid=(GRID,),
        in_specs=[
            pl.BlockSpec((MB, D), lambda i: (i, 0)),
            pl.BlockSpec((MB, D), lambda i: (i, 0)),
            pl.BlockSpec((MB, NW), lambda i: (i, 0)),
            pl.BlockSpec((1, D), lambda i: (0, 0)),
        ],
        out_specs=pl.BlockSpec((MB, D), lambda i: (i, 0)),
        out_shape=jax.ShapeDtypeStruct((N, D), jnp.float32),
    )(agg2, hs2, degpt, b2)


def kernel(x, edge_index, W1, b1, W2, b2):
    npad = E_PAD - E
    src_p = jnp.concatenate(
        [edge_index[0], jnp.zeros((npad,), edge_index.dtype)])
    dst_p = jnp.concatenate(
        [edge_index[1], jnp.full((npad,), N, edge_index.dtype)])
    dst32 = dst_p.reshape(NW, DWIN_N, WIN_K)
    src16 = src_p.reshape(NS, AWIN_N, WIN_K)
    dst16 = dst_p.reshape(NS, AWIN_N, WIN_K)
    zeros_rows = jnp.zeros((ACH, D), jnp.float32)
    zeros_reg = jnp.zeros((NREG,), jnp.float32)
    ones_w = jnp.ones((WIN_K,), jnp.float32)

    degp = _sc_degree(dst32, zeros_reg, ones_w)
    degpt = degp.reshape(NW, NREG)[:, :N].T

    hs1 = _tc_layer1(x, W1, degpt)
    agg1 = _sc_aggregate(hs1, src16, dst16, zeros_rows)
    hs2 = _tc_layer2(agg1, hs1, degpt, W2, b1.reshape(1, D))
    agg2 = _sc_aggregate(hs2, src16, dst16, zeros_rows)
    return _tc_layer3(agg2, hs2, degpt, b2.reshape(1, D))


# speedup vs baseline: 10.1872x; 1.2214x over previous
import functools

import jax
import jax.numpy as jnp
from jax import lax
from jax.experimental import pallas as pl
from jax.experimental.pallas import tpu as pltpu
from jax.experimental.pallas import tpu_sc as plsc

N = 10000
E = 320000
D = 128
NC = 2
NS = 16
NW = NC * NS
WIN_K = 128
E_PAD = 323584

DWIN_N = E_PAD // NW // WIN_K
NREG = N + 8

AWIN_N = E_PAD // NS // WIN_K
NH = N // NC
PARKN = 256
ACC_ROWS = 5280
ACH = 40

MB = 2000
GRID = N // MB

_MESH = dict(core_axis_name="c", subcore_axis_name="s",
             num_cores=NC, num_subcores=NS)


def _sc_degree(dst3, zeros_reg, ones_w):
    @functools.partial(
        pl.kernel,
        out_type=jax.ShapeDtypeStruct((NW * NREG,), jnp.float32),
        mesh=plsc.VectorSubcoreMesh(**_MESH),
        scratch_types=[
            pltpu.VMEM((DWIN_N, WIN_K), jnp.int32),
            pltpu.VMEM((WIN_K,), jnp.float32),
            pltpu.VMEM((NREG,), jnp.float32),
            pltpu.VMEM_SHARED((NS * NREG,), jnp.float32),
        ],
    )
    def deg_kernel(dst_hbm, z_hbm, ones_hbm, out_hbm, dstv, onesv, vbuf, cnt):
        c = lax.axis_index("c")
        s = lax.axis_index("s")
        w = c * NS + s
        pltpu.sync_copy(dst_hbm.at[w], dstv)
        pltpu.sync_copy(ones_hbm, onesv)

        base = s * NREG

        def remap(j, carry):
            for k in range(WIN_K // 16):
                dstv[j, pl.ds(k * 16, 16)] = (
                    dstv[j, pl.ds(k * 16, 16)] + base)
            return carry

        lax.fori_loop(0, DWIN_N, remap, 0)

        pltpu.sync_copy(z_hbm, vbuf)
        pltpu.sync_copy(vbuf, cnt.at[pl.ds(s * NREG, NREG)])

        def body(j, carry):
            pltpu.sync_copy(onesv, cnt.at[dstv.at[j]], add=True)
            return carry

        lax.fori_loop(0, DWIN_N, body, 0)
        pltpu.sync_copy(cnt.at[pl.ds(s * NREG, NREG)], vbuf)
        pltpu.sync_copy(vbuf, out_hbm.at[pl.ds(w * NREG, NREG)])

    return deg_kernel(dst3, zeros_reg, ones_w)


def _sc_aggregate(hs, src3, dst3, zeros_rows):
    @functools.partial(
        pl.kernel,
        out_type=jax.ShapeDtypeStruct((N, D), jnp.float32),
        mesh=plsc.VectorSubcoreMesh(**_MESH),
        scratch_types=[
            pltpu.VMEM((AWIN_N, WIN_K), jnp.int32),
            pltpu.VMEM((AWIN_N, WIN_K), jnp.int32),
            pltpu.VMEM((WIN_K, D), jnp.float32),
            pltpu.VMEM((WIN_K, D), jnp.float32),
            pltpu.VMEM((ACH, D), jnp.float32),
            pltpu.VMEM_SHARED((ACC_ROWS, D), jnp.float32),
            pltpu.SemaphoreType.DMA,
            pltpu.SemaphoreType.DMA,
        ],
    )
    def agg_kernel(hs_hbm, src_hbm, dst_hbm, z_hbm, out_hbm,
                   srcv, dstv, buf0, buf1, chbuf, acc, sem0, sem1):
        c = lax.axis_index("c")
        s = lax.axis_index("s")
        pltpu.sync_copy(src_hbm.at[s], srcv)
        pltpu.sync_copy(dst_hbm.at[s], dstv)

        lane = lax.iota(jnp.int32, 16)
        base = c * NH

        def remap(jj, carry):
            for m in range(2):
                j = jj * 2 + m
                for k in range(WIN_K // 16):
                    d = dstv[j, pl.ds(k * 16, 16)]
                    local = d - base
                    ok = (local >= 0) & (local < NH)
                    park = NH + m * WIN_K + k * 16 + lane
                    dstv[j, pl.ds(k * 16, 16)] = jnp.where(ok, local, park)
            return carry

        lax.fori_loop(0, AWIN_N // 2, remap, 0)

        pltpu.sync_copy(z_hbm, chbuf)
        for k in range(-(-(ACC_ROWS // ACH) // NS)):
            ch = s + k * NS

            @pl.when(ch < ACC_ROWS // ACH)
            def _():
                pltpu.sync_copy(chbuf, acc.at[pl.ds(ch * ACH, ACH)])

        plsc.subcore_barrier()

        pltpu.async_copy(hs_hbm.at[srcv.at[0]], buf0, sem0)

        def body(i, carry):
            j = i * 2
            pltpu.async_copy(hs_hbm.at[srcv.at[j + 1]], buf1, sem1)
            pltpu.make_async_copy(hs_hbm.at[srcv.at[j]], buf0, sem0).wait()
            pltpu.sync_copy(buf0, acc.at[dstv.at[j]], add=True)

            @pl.when(j + 2 < AWIN_N)
            def _():
                pltpu.async_copy(hs_hbm.at[srcv.at[j + 2]], buf0, sem0)

            pltpu.make_async_copy(
                hs_hbm.at[srcv.at[j + 1]], buf1, sem1).wait()
            pltpu.sync_copy(buf1, acc.at[dstv.at[j + 1]], add=True)
            return carry

        lax.fori_loop(0, AWIN_N // 2, body, 0)
        plsc.subcore_barrier()

        for k in range(-(-(NH // ACH) // NS)):
            ch = s + k * NS

            @pl.when(ch < NH // ACH)
            def _():
                pltpu.sync_copy(acc.at[pl.ds(ch * ACH, ACH)], chbuf)
                pltpu.sync_copy(
                    chbuf, out_hbm.at[pl.ds(c * NH + ch * ACH, ACH)])

    return agg_kernel(hs, src3, dst3, zeros_rows)


def _tc_layer1(x, W1, degpt):
    def body(x_ref, w_ref, dp_ref, hs_ref):
        g = lax.rsqrt(1.0 + jnp.sum(dp_ref[...], axis=1, keepdims=True))
        hs_ref[...] = (
            jnp.dot(x_ref[...], w_ref[...], preferred_element_type=jnp.float32)
            * g
        )

    return pl.pallas_call(
        body,
        grid=(GRID,),
        in_specs=[
            pl.BlockSpec((MB, D), lambda i: (i, 0)),
            pl.BlockSpec((D, D), lambda i: (0, 0)),
            pl.BlockSpec((MB, NW), lambda i: (i, 0)),
        ],
        out_specs=pl.BlockSpec((MB, D), lambda i: (i, 0)),
        out_shape=jax.ShapeDtypeStruct((N, D), jnp.float32),
    )(x, W1, degpt)


def _tc_layer2(agg1, hs1, degpt, W2, b1):
    def body(a_ref, h_ref, dp_ref, w_ref, b_ref, out_ref):
        g = lax.rsqrt(1.0 + jnp.sum(dp_ref[...], axis=1, keepdims=True))
        t = (a_ref[...] + h_ref[...]) * g + b_ref[...]
        t = jnp.maximum(t, 0.0)
        out_ref[...] = (
            jnp.dot(t, w_ref[...], preferred_element_type=jnp.float32) * g
        )

    return pl.pallas_call(
        body,
        grid=(GRID,),
        in_specs=[
            pl.BlockSpec((MB, D), lambda i: (i, 0)),
            pl.BlockSpec((MB, D), lambda i: (i, 0)),
            pl.BlockSpec((MB, NW), lambda i: (i, 0)),
            pl.BlockSpec((D, D), lambda i: (0, 0)),
            pl.BlockSpec((1, D), lambda i: (0, 0)),
        ],
        out_specs=pl.BlockSpec((MB, D), lambda i: (i, 0)),
        out_shape=jax.ShapeDtypeStruct((N, D), jnp.float32),
    )(agg1, hs1, degpt, W2, b1)


def _tc_layer3(agg2, hs2, degpt, b2):
    def body(a_ref, h_ref, dp_ref, b_ref, out_ref):
        g = lax.rsqrt(1.0 + jnp.sum(dp_ref[...], axis=1, keepdims=True))
        out_ref[...] = (a_ref[...] + h_ref[...]) * g + b_ref[...]

    return pl.pallas_call(
        body,
        grid=(GRID,),
        in_specs=[
            pl.BlockSpec((MB, D), lambda i: (i, 0)),
            pl.BlockSpec((MB, D), lambda i: (i, 0)),
            pl.BlockSpec((MB, NW), lambda i: (i, 0)),
            pl.BlockSpec((1, D), lambda i: (0, 0)),
        ],
        out_specs=pl.BlockSpec((MB, D), lambda i: (i, 0)),
        out_shape=jax.ShapeDtypeStruct((N, D), jnp.float32),
    )(agg2, hs2, degpt, b2)


def kernel(x, edge_index, W1, b1, W2, b2):
    npad = E_PAD - E
    src_p = jnp.concatenate(
        [edge_index[0], jnp.zeros((npad,), edge_index.dtype)])
    dst_p = jnp.concatenate(
        [edge_index[1], jnp.full((npad,), N, edge_index.dtype)])
    dst32 = dst_p.reshape(NW, DWIN_N, WIN_K)
    src16 = src_p.reshape(NS, AWIN_N, WIN_K)
    dst16 = dst_p.reshape(NS, AWIN_N, WIN_K)
    zeros_rows = jnp.zeros((ACH, D), jnp.float32)
    zeros_reg = jnp.zeros((NREG,), jnp.float32)
    ones_w = jnp.ones((WIN_K,), jnp.float32)

    degp = _sc_degree(dst32, zeros_reg, ones_w)
    degpt = degp.reshape(NW, NREG)[:, :N].T

    hs1 = _tc_layer1(x, W1, degpt)
    agg1 = _sc_aggregate(hs1, src16, dst16, zeros_rows)
    hs2 = _tc_layer2(agg1, hs1, degpt, W2, b1.reshape(1, D))
    agg2 = _sc_aggregate(hs2, src16, dst16, zeros_rows)
    return _tc_layer3(agg2, hs2, degpt, b2.reshape(1, D))
